# baseline (speedup 1.0000x reference)
import functools

import jax
import jax.numpy as jnp
from jax import lax
from jax.experimental import pallas as pl
from jax.experimental.pallas import tpu as pltpu
from jax.experimental.pallas import tpu_sc as plsc

P = 100000
F = 100
B = 16384
NF = 50
ROW = F * 3
PROW = 2 * ROW
OROW = NF * 3
NW = 32
PPW = B // NW
C = 32
NCHUNK = PPW // C
L = 16

_MAGIC = 0x5F3759DF


def _rsqrt(s):
    i = plsc.bitcast(s, jnp.int32)
    i = _MAGIC - lax.shift_right_logical(i, 1)
    r = plsc.bitcast(i, jnp.float32)
    for _ in range(3):
        r = r * (1.5 - 0.5 * s * r * r)
    return r


def _sc_kernel(tt_hbm, tn_hbm, frames_hbm, points_hbm,
               t_out_hbm, n_out_hbm,
               pts_v, idx_v, frames_v,
               st0, st1, sn0, sn1, ot_v, on_v,
               sem_t0, sem_t1, sem_n0, sem_n1):
    wid = lax.axis_index("s") * 2 + lax.axis_index("c")
    base = wid * PPW
    iota = lax.iota(jnp.int32, L)

    pltpu.sync_copy(points_hbm.at[pl.ds(base, PPW)], pts_v)
    for g in range(NCHUNK):
        for k in range(C // L):
            p16 = pts_v[pl.ds(g * C + k * L, L)]
            idx_v[g, pl.ds(k * L, L)] = lax.shift_right_logical(p16, 1)

    pltpu.sync_copy(frames_hbm, frames_v.at[pl.ds(0, NF)])

    stage_t = (st0, st1)
    stage_n = (sn0, sn1)
    sems_t = (sem_t0, sem_t1)
    sems_n = (sem_n0, sem_n1)

    def start(g):
        b = g % 2
        ht = pltpu.async_copy(tt_hbm.at[idx_v.at[g]], stage_t[b], sems_t[b])
        hn = pltpu.async_copy(tn_hbm.at[idx_v.at[g]], stage_n[b], sems_n[b])
        return ht, hn

    def compute(g):
        b = g % 2
        st, sn = stage_t[b], stage_n[b]
        for k in range(C // L):
            p16 = pts_v[pl.ds(g * C + k * L, L)]
            par = (p16 & 1) * ROW
            row16 = iota + k * L
            obase = (iota + k * L) * OROW

            def body(j, _):
                fvec = frames_v[pl.ds(j, L)]
                fj3 = fvec[0] * 3
                col = par + fj3
                oidx = obase + j * 3
                for c in range(3):
                    v = plsc.load_gather(st, [row16, col + c])
                    plsc.store_scatter(ot_v, [oidx + c], v)
                x = plsc.load_gather(sn, [row16, col])
                y = plsc.load_gather(sn, [row16, col + 1])
                z = plsc.load_gather(sn, [row16, col + 2])
                s = x * x + y * y + z * z
                r = _rsqrt(jnp.maximum(s, 1e-24))
                plsc.store_scatter(on_v, [oidx], x * r)
                plsc.store_scatter(on_v, [oidx + 1], y * r)
                plsc.store_scatter(on_v, [oidx + 2], z * r)
                return _

            lax.fori_loop(0, NF, body, None)

    handles = {}
    handles[0] = start(0)
    for g in range(NCHUNK):
        if g + 1 < NCHUNK:
            handles[g + 1] = start(g + 1)
        ht, hn = handles.pop(g)
        ht.wait()
        hn.wait()
        compute(g)
        obase = (base + g * C) * OROW
        pltpu.sync_copy(ot_v, t_out_hbm.at[pl.ds(obase, C * OROW)])
        pltpu.sync_copy(on_v, n_out_hbm.at[pl.ds(obase, C * OROW)])


@jax.jit
def _run(tt, tn, frames, points):
    mesh = plsc.VectorSubcoreMesh(core_axis_name="c", subcore_axis_name="s")
    out = jax.ShapeDtypeStruct((B * OROW,), jnp.float32)
    k = functools.partial(
        pl.kernel,
        mesh=mesh,
        out_type=(out, out),
        compiler_params=pltpu.CompilerParams(use_tc_tiling_on_sc=False,
                                             needs_layout_passes=False),
        scratch_types=[
            pltpu.VMEM((PPW,), jnp.int32),
            pltpu.VMEM((NCHUNK, C), jnp.int32),
            pltpu.VMEM((NF + L,), jnp.int32),
            pltpu.VMEM((C, PROW), jnp.float32),
            pltpu.VMEM((C, PROW), jnp.float32),
            pltpu.VMEM((C, PROW), jnp.float32),
            pltpu.VMEM((C, PROW), jnp.float32),
            pltpu.VMEM((C * OROW,), jnp.float32),
            pltpu.VMEM((C * OROW,), jnp.float32),
            pltpu.SemaphoreType.DMA,
            pltpu.SemaphoreType.DMA,
            pltpu.SemaphoreType.DMA,
            pltpu.SemaphoreType.DMA,
        ],
    )(_sc_kernel)
    t_flat, n_flat = k(tt, tn, frames, points)
    return t_flat.reshape(B, NF, 3), n_flat.reshape(B, NF, 3)


def kernel(delta_translation, delta_normal, frames, points):
    tt = delta_translation.reshape(P // 2, PROW)
    tn = delta_normal.reshape(P // 2, PROW)
    return _run(tt, tn, frames.astype(jnp.int32), points.astype(jnp.int32))
